# baseline (speedup 1.0000x reference)
import functools
import jax
import jax.numpy as jnp
from jax import lax
from jax.experimental import pallas as pl
from jax.experimental.pallas import tpu as pltpu
from jax.experimental.pallas import tpu_sc as plsc

B = 1024
L = 200
DIM = 64
NTOK = B * L
NW = 32
TOK_PER_W = NTOK // NW
CH = 128
NCH = TOK_PER_W // CH

_scratch_types = [
    pltpu.VMEM((B + 16,), jnp.int32),
    pltpu.VMEM((CH,), jnp.int32),
    pltpu.VMEM((CH,), jnp.int32),
    pltpu.VMEM((CH,), jnp.int32),
    pltpu.VMEM((CH,), jnp.int32),
    pltpu.VMEM((CH, DIM), jnp.float32),
    pltpu.VMEM((CH, DIM), jnp.float32),
    pltpu.VMEM((CH, DIM), jnp.float32),
    pltpu.SemaphoreType.DMA,
    pltpu.SemaphoreType.DMA,
]


def _emb_body(ids_hbm, voc_hbm, len_hbm, llm_hbm, cod_hbm, out_hbm,
              len_v, ids_v, voc_v, lidx_v, cidx_v,
              lrows_v, crows_v, obuf_v, sem1, sem2):
    wid = lax.axis_index("s") * 2 + lax.axis_index("c")
    pltpu.sync_copy(len_hbm, len_v.at[pl.ds(0, B)])

    def chunk_body(c, carry):
        base = wid * TOK_PER_W + c * CH
        pltpu.sync_copy(ids_hbm.at[pl.ds(base, CH)], ids_v)
        pltpu.sync_copy(voc_hbm.at[pl.ds(base, CH)], voc_v)
        b_lo = lax.div(base, jnp.int32(L))
        b_hi = lax.div(base + (CH - 1), jnp.int32(L))
        l0 = base - b_lo * jnp.int32(L)
        e = jnp.minimum(jnp.int32(L) - l0, CH)
        n_lo = len_v[pl.ds(b_lo, 16)][0]
        n_hi = len_v[pl.ds(b_hi, 16)][0]
        s1 = n_lo - l0
        s2 = e + n_hi

        def build(i, carry2):
            sl = pl.ds(i * 16, 16)
            t16 = i * 16 + lax.iota(jnp.int32, 16)
            valid = jnp.where(t16 < e, t16 < s1, t16 < s2)
            ids16 = ids_v[sl]
            v16 = voc_v[sl]
            zero = jnp.zeros((16,), jnp.int32)
            lidx_v[sl] = jnp.where(valid & (v16 == 0), ids16, zero)
            cidx_v[sl] = jnp.where(valid & (v16 == 1), ids16, zero)
            return carry2

        lax.fori_loop(0, CH // 16, build, 0)

        cp1 = pltpu.async_copy(llm_hbm.at[lidx_v], lrows_v, sem1)
        cp2 = pltpu.async_copy(cod_hbm.at[cidx_v], crows_v, sem2)
        cp1.wait()
        cp2.wait()

        def comb(t, carry3):
            for d in range(DIM // 16):
                sl = pl.ds(d * 16, 16)
                obuf_v[t, sl] = lrows_v[t, sl] + crows_v[t, sl]
            return carry3

        lax.fori_loop(0, CH, comb, 0)

        zero16 = jnp.zeros((16,), jnp.float32)

        def zfill(t, carry4):
            for d in range(DIM // 16):
                obuf_v[t, pl.ds(d * 16, 16)] = zero16
            return carry4

        lax.fori_loop(jnp.clip(s1, 0, e), e, zfill, 0)
        lax.fori_loop(jnp.clip(s2, e, CH), CH, zfill, 0)

        pltpu.sync_copy(obuf_v, out_hbm.at[pl.ds(base, CH)])
        return carry

    lax.fori_loop(0, NCH, chunk_body, 0)


@functools.cache
def _build_kernel():
    mesh = plsc.VectorSubcoreMesh(
        core_axis_name="c", subcore_axis_name="s", num_cores=2, num_subcores=16
    )
    return pl.kernel(
        _emb_body,
        out_type=jax.ShapeDtypeStruct((NTOK, DIM), jnp.float32),
        mesh=mesh,
        scratch_types=_scratch_types,
        compiler_params=pltpu.CompilerParams(use_tc_tiling_on_sc=False),
    )


def kernel(input_ids, vocab_ids, length, llm_table, cod_table):
    ids = input_ids.reshape(-1).astype(jnp.int32)
    voc = vocab_ids.reshape(-1).astype(jnp.int32)
    ln = length.astype(jnp.int32)
    out = _build_kernel()(ids, voc, ln, llm_table, cod_table)
    emb = out.reshape(B, L, DIM)
    attention_mask = jnp.arange(L)[None, :] < length[:, None]
    return (emb, attention_mask)
